# baseline (speedup 1.0000x reference)
import jax
import jax.numpy as jnp
from jax.experimental import pallas as pl
from jax.experimental.pallas import tpu as pltpu

B, S = 1, 2048
D = 1024
H, KH, HD = 16, 4, 64
E, TOPK, F = 8, 2, 512
EPS = 1e-05
THETA = 10000.0
ALPHA = 1.702
LIMIT = 7.0
SCALE = HD ** -0.5
QW = H * HD
KW = KH * HD

BT1 = 512
BQ = 256
KB = 256
BT3 = 512
BT4 = 512

_NEG = -1e30


def _row_sum(x):
    n = x.shape[-1] // 128
    v = x[:, 0:128]
    for j in range(1, n):
        v = v + x[:, 128 * j:128 * (j + 1)]
    s = v[:, 0:8]
    for g in range(1, 16):
        s = s + v[:, 8 * g:8 * (g + 1)]
    s = s[:, :4] + s[:, 4:]
    s = s[:, :2] + s[:, 2:]
    return s[:, :1] + s[:, 1:]


def _qkv_body(cos_ref, sin_ref, x_ref, r_ref, w1_ref, qkvw_ref, qkvb_ref,
              q_ref, k_ref, v_ref, nr_ref):
    x = x_ref[...] + r_ref[...]
    nr_ref[...] = x
    var = _row_sum(x * x) * (1.0 / D)
    h = x * jax.lax.rsqrt(var + EPS) * w1_ref[...]
    qkv = jnp.dot(h, qkvw_ref[...], preferred_element_type=jnp.float32)
    qkv = qkv + qkvb_ref[...]
    c = cos_ref[...]
    s = sin_ref[...]

    def rope(xh):
        x1 = xh[:, : HD // 2]
        x2 = xh[:, HD // 2:]
        return jnp.concatenate([x1 * c - x2 * s, x2 * c + x1 * s], axis=-1)

    for hh in range(H):
        q_ref[hh] = rope(qkv[:, hh * HD:(hh + 1) * HD])
    for hh in range(KH):
        k_ref[hh] = rope(qkv[:, QW + hh * HD: QW + (hh + 1) * HD])
        v_ref[hh] = qkv[:, QW + KW + hh * HD: QW + KW + (hh + 1) * HD]


def _attn_body(sink_ref, q_ref, k_ref, v_ref, o_ref, s_ref):
    h = pl.program_id(0)
    qi = pl.program_id(1)
    q = q_ref[0]
    lane = jax.lax.broadcasted_iota(jnp.int32, (1, H), 1)
    sink = jnp.sum(jnp.where(lane == h, sink_ref[...], 0.0))
    row = qi * BQ + jax.lax.broadcasted_iota(jnp.int32, (BQ, 1), 0)
    nk = qi + 1

    def pass1(j, m):
        k = k_ref[0, pl.ds(j * KB, KB), :]
        sc = jax.lax.dot_general(q, k, (((1,), (1,)), ((), ())),
                                 preferred_element_type=jnp.float32) * SCALE
        col = j * KB + jax.lax.broadcasted_iota(jnp.int32, (1, KB), 1)
        sc = jnp.where(row >= col, sc, -jnp.inf)
        s_ref[:, pl.ds(j * KB, KB)] = sc
        return jnp.maximum(m, jnp.max(sc, axis=-1, keepdims=True))

    m0 = jnp.full((BQ, 1), sink, jnp.float32)
    m = jax.lax.fori_loop(0, nk, pass1, m0)

    def pass2(j, v):
        e = jnp.exp(s_ref[:, pl.ds(j * KB, KB)] - m)
        s_ref[:, pl.ds(j * KB, KB)] = e
        v = v + e[:, 0:128]
        return v + e[:, 128:]

    v = jax.lax.fori_loop(0, nk, pass2, jnp.zeros((BQ, 128), jnp.float32))
    ek = jnp.exp(sink - m)
    lane0 = jax.lax.broadcasted_iota(jnp.int32, (1, 128), 1) == 0
    v = v + jnp.where(lane0, ek, 0.0)
    s8 = v[:, 0:8]
    for g in range(1, 16):
        s8 = s8 + v[:, 8 * g:8 * (g + 1)]
    s8 = s8[:, :4] + s8[:, 4:]
    s8 = s8[:, :2] + s8[:, 2:]
    l = s8[:, :1] + s8[:, 1:]
    rl = 1.0 / l

    def pass3(j, acc):
        p = s_ref[:, pl.ds(j * KB, KB)] * rl
        vv = v_ref[0, pl.ds(j * KB, KB), :]
        return acc + jnp.dot(p, vv, preferred_element_type=jnp.float32)

    acc = jax.lax.fori_loop(0, nk, pass3, jnp.zeros((BQ, HD), jnp.float32))
    o_ref[0] = acc


def _post_body(ao_ref, r_ref, w2_ref, ow_ref, ob_ref, rw_ref, rb_ref,
               x_ref, nr_ref, sc_ref):
    ao = jnp.concatenate([ao_ref[hh] for hh in range(H)], axis=-1)
    o = jnp.dot(ao, ow_ref[...], preferred_element_type=jnp.float32)
    x = o + ob_ref[...] + r_ref[...]
    nr_ref[...] = x
    var = _row_sum(x * x) * (1.0 / D)
    xn = x * jax.lax.rsqrt(var + EPS) * w2_ref[...]
    x_ref[...] = xn
    logits = jnp.dot(xn, rw_ref[...], preferred_element_type=jnp.float32)
    logits = logits + rb_ref[...]
    idx = jax.lax.broadcasted_iota(jnp.int32, (1, 128), 1)
    m1 = jnp.max(logits, axis=-1, keepdims=True)
    eq1 = logits == m1
    am1 = 127 - jnp.max(jnp.where(eq1, 127 - idx, -1), axis=-1, keepdims=True)
    rest = jnp.where(idx == am1, _NEG, logits)
    m2 = jnp.max(rest, axis=-1, keepdims=True)
    eq2 = rest == m2
    am2 = 127 - jnp.max(jnp.where(eq2, 127 - idx, -1), axis=-1, keepdims=True)
    e2 = jnp.exp(m2 - m1)
    rec = 1.0 / (1.0 + e2)
    w1 = rec
    w2 = e2 * rec
    sc_ref[...] = jnp.where(idx == am1, w1, 0.0) + jnp.where(idx == am2, w2, 0.0)


def _moe_body(x_ref, sc_ref, guw_ref, gub_ref, dw_ref, db_ref, o_ref):
    e = pl.program_id(1)
    x = x_ref[...]
    gu = jnp.dot(x, guw_ref[0], preferred_element_type=jnp.float32)
    gu = gu + gub_ref[0]
    gate = jnp.minimum(gu[:, :F], LIMIT)
    up = jnp.clip(gu[:, F:], -LIMIT, LIMIT)
    glu = gate * jax.nn.sigmoid(gate * ALPHA)
    hmid = (up + 1.0) * glu
    eo = jnp.dot(hmid, dw_ref[0], preferred_element_type=jnp.float32)
    eo = eo + db_ref[0]
    lane = jax.lax.broadcasted_iota(jnp.int32, (1, 128), 1)
    w = jnp.sum(jnp.where(lane == e, sc_ref[...], 0.0), axis=-1, keepdims=True)
    contrib = eo * w

    @pl.when(e == 0)
    def _():
        o_ref[...] = contrib

    @pl.when(e != 0)
    def _():
        o_ref[...] += contrib


def _forward(hidden_states, residual, positions, ln1_w, ln2_w, qkv_w, qkv_b,
             o_w, o_b, sinks, router_w, router_b, gate_up_w, gate_up_b,
             down_w, down_b):
    x = hidden_states.reshape(S, D)
    r = residual.reshape(S, D)
    inv = 1.0 / (THETA ** (jnp.arange(0, HD, 2, dtype=jnp.float32) / HD))
    freqs = positions.astype(jnp.float32).reshape(S, 1) * inv
    cos_t = jnp.cos(freqs)
    sin_t = jnp.sin(freqs)
    ln1 = ln1_w.reshape(1, D)
    ln2 = ln2_w.reshape(1, D)
    qkv_b2 = qkv_b.reshape(1, (H + 2 * KH) * HD)
    o_b2 = o_b.reshape(1, D)
    sinks2 = sinks.reshape(1, H)
    rw_p = jnp.zeros((D, 128), jnp.float32).at[:, :E].set(router_w)
    rb_p = jnp.full((1, 128), _NEG, jnp.float32).at[0, :E].set(router_b)
    guw = jnp.concatenate([gate_up_w[:, :, 0::2], gate_up_w[:, :, 1::2]],
                          axis=-1)
    gub = jnp.concatenate([gate_up_b[:, 0::2], gate_up_b[:, 1::2]],
                          axis=-1).reshape(E, 1, 2 * F)
    db3 = down_b.reshape(E, 1, D)

    n1 = S // BT1
    q_rot, k_rot, v_all, resid1 = pl.pallas_call(
        _qkv_body,
        grid=(n1,),
        in_specs=[
            pl.BlockSpec((BT1, HD // 2), lambda i: (i, 0)),
            pl.BlockSpec((BT1, HD // 2), lambda i: (i, 0)),
            pl.BlockSpec((BT1, D), lambda i: (i, 0)),
            pl.BlockSpec((BT1, D), lambda i: (i, 0)),
            pl.BlockSpec((1, D), lambda i: (0, 0)),
            pl.BlockSpec((D, (H + 2 * KH) * HD), lambda i: (0, 0)),
            pl.BlockSpec((1, (H + 2 * KH) * HD), lambda i: (0, 0)),
        ],
        out_specs=[
            pl.BlockSpec((H, BT1, HD), lambda i: (0, i, 0)),
            pl.BlockSpec((KH, BT1, HD), lambda i: (0, i, 0)),
            pl.BlockSpec((KH, BT1, HD), lambda i: (0, i, 0)),
            pl.BlockSpec((BT1, D), lambda i: (i, 0)),
        ],
        out_shape=[
            jax.ShapeDtypeStruct((H, S, HD), jnp.float32),
            jax.ShapeDtypeStruct((KH, S, HD), jnp.float32),
            jax.ShapeDtypeStruct((KH, S, HD), jnp.float32),
            jax.ShapeDtypeStruct((S, D), jnp.float32),
        ],
    )(cos_t, sin_t, x, r, ln1, qkv_w, qkv_b2)

    nq = S // BQ
    attn = pl.pallas_call(
        _attn_body,
        grid=(H, nq),
        in_specs=[
            pl.BlockSpec((1, H), lambda h, qi: (0, 0)),
            pl.BlockSpec((1, BQ, HD), lambda h, qi: (h, qi, 0)),
            pl.BlockSpec((1, S, HD), lambda h, qi: (h // 4, 0, 0)),
            pl.BlockSpec((1, S, HD), lambda h, qi: (h // 4, 0, 0)),
        ],
        out_specs=pl.BlockSpec((1, BQ, HD), lambda h, qi: (h, qi, 0)),
        out_shape=jax.ShapeDtypeStruct((H, S, HD), jnp.float32),
        scratch_shapes=[pltpu.VMEM((BQ, S), jnp.float32)],
    )(sinks2, q_rot, k_rot, v_all)

    n3 = S // BT3
    xflat, resid2, scores = pl.pallas_call(
        _post_body,
        grid=(n3,),
        in_specs=[
            pl.BlockSpec((H, BT3, HD), lambda i: (0, i, 0)),
            pl.BlockSpec((BT3, D), lambda i: (i, 0)),
            pl.BlockSpec((1, D), lambda i: (0, 0)),
            pl.BlockSpec((QW, D), lambda i: (0, 0)),
            pl.BlockSpec((1, D), lambda i: (0, 0)),
            pl.BlockSpec((D, 128), lambda i: (0, 0)),
            pl.BlockSpec((1, 128), lambda i: (0, 0)),
        ],
        out_specs=[
            pl.BlockSpec((BT3, D), lambda i: (i, 0)),
            pl.BlockSpec((BT3, D), lambda i: (i, 0)),
            pl.BlockSpec((BT3, 128), lambda i: (i, 0)),
        ],
        out_shape=[
            jax.ShapeDtypeStruct((S, D), jnp.float32),
            jax.ShapeDtypeStruct((S, D), jnp.float32),
            jax.ShapeDtypeStruct((S, 128), jnp.float32),
        ],
    )(attn, resid1, ln2, o_w, o_b2, rw_p, rb_p)

    n4 = S // BT4
    out = pl.pallas_call(
        _moe_body,
        grid=(n4, E),
        in_specs=[
            pl.BlockSpec((BT4, D), lambda t, e: (t, 0)),
            pl.BlockSpec((BT4, 128), lambda t, e: (t, 0)),
            pl.BlockSpec((1, D, 2 * F), lambda t, e: (e, 0, 0)),
            pl.BlockSpec((1, 1, 2 * F), lambda t, e: (e, 0, 0)),
            pl.BlockSpec((1, F, D), lambda t, e: (e, 0, 0)),
            pl.BlockSpec((1, 1, D), lambda t, e: (e, 0, 0)),
        ],
        out_specs=pl.BlockSpec((BT4, D), lambda t, e: (t, 0)),
        out_shape=jax.ShapeDtypeStruct((S, D), jnp.float32),
    )(xflat, scores, guw, gub, down_w, db3)

    return dict(out=out, resid2=resid2, xflat=xflat, scores=scores,
                attn=attn, resid1=resid1, q_rot=q_rot, k_rot=k_rot)


def kernel(hidden_states, residual, positions, ln1_w, ln2_w, qkv_w, qkv_b,
           o_w, o_b, sinks, router_w, router_b, gate_up_w, gate_up_b,
           down_w, down_b):
    p = _forward(hidden_states, residual, positions, ln1_w, ln2_w, qkv_w,
                 qkv_b, o_w, o_b, sinks, router_w, router_b, gate_up_w,
                 gate_up_b, down_w, down_b)
    return p["out"].reshape(B, S, D), p["resid2"].reshape(B, S, D)


# speedup vs baseline: 1.1330x; 1.1330x over previous
import jax
import jax.numpy as jnp
from jax.experimental import pallas as pl
from jax.experimental.pallas import tpu as pltpu

B, S = 1, 2048
D = 1024
H, KH, HD = 16, 4, 64
E, TOPK, F = 8, 2, 512
EPS = 1e-05
THETA = 10000.0
ALPHA = 1.702
LIMIT = 7.0
SCALE = HD ** -0.5
QW = H * HD
KW = KH * HD

BT1 = 512
BQ = 512
KB = 512
BT3 = 512
BT4 = 1024

_NEG = -1e30


def _row_sum(x):
    n = x.shape[-1] // 128
    v = x[:, 0:128]
    for j in range(1, n):
        v = v + x[:, 128 * j:128 * (j + 1)]
    s = v[:, 0:8]
    for g in range(1, 16):
        s = s + v[:, 8 * g:8 * (g + 1)]
    s = s[:, :4] + s[:, 4:]
    s = s[:, :2] + s[:, 2:]
    return s[:, :1] + s[:, 1:]


def _qkv_body(cos_ref, sin_ref, x_ref, r_ref, w1_ref, qkvw_ref, qkvb_ref,
              q_ref, k_ref, v_ref, nr_ref):
    x = x_ref[...] + r_ref[...]
    nr_ref[...] = x
    var = _row_sum(x * x) * (1.0 / D)
    h = x * jax.lax.rsqrt(var + EPS) * w1_ref[...]
    qkv = jnp.dot(h, qkvw_ref[...], preferred_element_type=jnp.float32)
    qkv = qkv + qkvb_ref[...]
    c = cos_ref[...]
    s = sin_ref[...]

    def rope(xh):
        x1 = xh[:, : HD // 2]
        x2 = xh[:, HD // 2:]
        return jnp.concatenate([x1 * c - x2 * s, x2 * c + x1 * s], axis=-1)

    for hh in range(H):
        q_ref[hh] = rope(qkv[:, hh * HD:(hh + 1) * HD])
    for hh in range(KH):
        k_ref[hh] = rope(qkv[:, QW + hh * HD: QW + (hh + 1) * HD])
        v_ref[hh] = qkv[:, QW + KW + hh * HD: QW + KW + (hh + 1) * HD]


def _attn_body(sink_ref, q_ref, k_ref, v_ref, o_ref, s_ref):
    h = pl.program_id(0)
    qi = pl.program_id(1)
    q = q_ref[0]
    lane = jax.lax.broadcasted_iota(jnp.int32, (1, H), 1)
    sink = jnp.sum(jnp.where(lane == h, sink_ref[...], 0.0))
    row = qi * BQ + jax.lax.broadcasted_iota(jnp.int32, (BQ, 1), 0)
    s_ref[...] = jnp.full((BQ, S), -jnp.inf, jnp.float32)

    def pass1(j, m):
        k = k_ref[0, pl.ds(j * KB, KB), :]
        sc = jax.lax.dot_general(q, k, (((1,), (1,)), ((), ())),
                                 preferred_element_type=jnp.float32) * SCALE
        col = j * KB + jax.lax.broadcasted_iota(jnp.int32, (1, KB), 1)
        sc = jnp.where(row >= col, sc, -jnp.inf)
        s_ref[:, pl.ds(j * KB, KB)] = sc
        return jnp.maximum(m, jnp.max(sc, axis=-1, keepdims=True))

    nk = (qi * BQ + BQ) // KB
    m0 = jnp.full((BQ, 1), sink, jnp.float32)
    m = jax.lax.fori_loop(0, nk, pass1, m0)
    e = jnp.exp(s_ref[...] - m)
    ek = jnp.exp(sink - m)
    v = e[:, 0:128]
    for j in range(1, S // 128):
        v = v + e[:, 128 * j:128 * (j + 1)]
    lane0 = jax.lax.broadcasted_iota(jnp.int32, (1, 128), 1) == 0
    v = v + jnp.where(lane0, ek, 0.0)
    s8 = v[:, 0:8]
    for g in range(1, 16):
        s8 = s8 + v[:, 8 * g:8 * (g + 1)]
    s8 = s8[:, :4] + s8[:, 4:]
    s8 = s8[:, :2] + s8[:, 2:]
    l = s8[:, :1] + s8[:, 1:]
    probs = e * (1.0 / l)
    o_ref[0] = jnp.dot(probs, v_ref[0], preferred_element_type=jnp.float32)


def _post_body(ao_ref, r_ref, w2_ref, ow_ref, ob_ref, rw_ref, rb_ref,
               x_ref, nr_ref, sc_ref):
    ao = jnp.concatenate([ao_ref[hh] for hh in range(H)], axis=-1)
    o = jnp.dot(ao, ow_ref[...], preferred_element_type=jnp.float32)
    x = o + ob_ref[...] + r_ref[...]
    nr_ref[...] = x
    var = _row_sum(x * x) * (1.0 / D)
    xn = x * jax.lax.rsqrt(var + EPS) * w2_ref[...]
    x_ref[...] = xn
    logits = jnp.dot(xn, rw_ref[...], preferred_element_type=jnp.float32)
    logits = logits + rb_ref[...]
    idx = jax.lax.broadcasted_iota(jnp.int32, (1, 128), 1)
    m1 = jnp.max(logits, axis=-1, keepdims=True)
    eq1 = logits == m1
    am1 = 127 - jnp.max(jnp.where(eq1, 127 - idx, -1), axis=-1, keepdims=True)
    rest = jnp.where(idx == am1, _NEG, logits)
    m2 = jnp.max(rest, axis=-1, keepdims=True)
    eq2 = rest == m2
    am2 = 127 - jnp.max(jnp.where(eq2, 127 - idx, -1), axis=-1, keepdims=True)
    e2 = jnp.exp(m2 - m1)
    rec = 1.0 / (1.0 + e2)
    w1 = rec
    w2 = e2 * rec
    sc_ref[...] = jnp.where(idx == am1, w1, 0.0) + jnp.where(idx == am2, w2, 0.0)


def _moe_body(x_ref, sc_ref, guw_ref, gub_ref, dw_ref, db_ref, o_ref):
    e = pl.program_id(1)
    x = x_ref[...]
    gu = jnp.dot(x, guw_ref[0], preferred_element_type=jnp.float32)
    gu = gu + gub_ref[0]
    gate = jnp.minimum(gu[:, :F], LIMIT)
    up = jnp.clip(gu[:, F:], -LIMIT, LIMIT)
    glu = gate * jax.nn.sigmoid(gate * ALPHA)
    hmid = (up + 1.0) * glu
    eo = jnp.dot(hmid, dw_ref[0], preferred_element_type=jnp.float32)
    eo = eo + db_ref[0]
    lane = jax.lax.broadcasted_iota(jnp.int32, (1, 128), 1)
    w = jnp.sum(jnp.where(lane == e, sc_ref[...], 0.0), axis=-1, keepdims=True)
    contrib = eo * w

    @pl.when(e == 0)
    def _():
        o_ref[...] = contrib

    @pl.when(e != 0)
    def _():
        o_ref[...] += contrib


def _forward(hidden_states, residual, positions, ln1_w, ln2_w, qkv_w, qkv_b,
             o_w, o_b, sinks, router_w, router_b, gate_up_w, gate_up_b,
             down_w, down_b):
    x = hidden_states.reshape(S, D)
    r = residual.reshape(S, D)
    inv = 1.0 / (THETA ** (jnp.arange(0, HD, 2, dtype=jnp.float32) / HD))
    freqs = positions.astype(jnp.float32).reshape(S, 1) * inv
    cos_t = jnp.cos(freqs)
    sin_t = jnp.sin(freqs)
    ln1 = ln1_w.reshape(1, D)
    ln2 = ln2_w.reshape(1, D)
    qkv_b2 = qkv_b.reshape(1, (H + 2 * KH) * HD)
    o_b2 = o_b.reshape(1, D)
    sinks2 = sinks.reshape(1, H)
    rw_p = jnp.zeros((D, 128), jnp.float32).at[:, :E].set(router_w)
    rb_p = jnp.full((1, 128), _NEG, jnp.float32).at[0, :E].set(router_b)
    guw = jnp.concatenate([gate_up_w[:, :, 0::2], gate_up_w[:, :, 1::2]],
                          axis=-1)
    gub = jnp.concatenate([gate_up_b[:, 0::2], gate_up_b[:, 1::2]],
                          axis=-1).reshape(E, 1, 2 * F)
    db3 = down_b.reshape(E, 1, D)

    n1 = S // BT1
    q_rot, k_rot, v_all, resid1 = pl.pallas_call(
        _qkv_body,
        grid=(n1,),
        in_specs=[
            pl.BlockSpec((BT1, HD // 2), lambda i: (i, 0)),
            pl.BlockSpec((BT1, HD // 2), lambda i: (i, 0)),
            pl.BlockSpec((BT1, D), lambda i: (i, 0)),
            pl.BlockSpec((BT1, D), lambda i: (i, 0)),
            pl.BlockSpec((1, D), lambda i: (0, 0)),
            pl.BlockSpec((D, (H + 2 * KH) * HD), lambda i: (0, 0)),
            pl.BlockSpec((1, (H + 2 * KH) * HD), lambda i: (0, 0)),
        ],
        out_specs=[
            pl.BlockSpec((H, BT1, HD), lambda i: (0, i, 0)),
            pl.BlockSpec((KH, BT1, HD), lambda i: (0, i, 0)),
            pl.BlockSpec((KH, BT1, HD), lambda i: (0, i, 0)),
            pl.BlockSpec((BT1, D), lambda i: (i, 0)),
        ],
        out_shape=[
            jax.ShapeDtypeStruct((H, S, HD), jnp.float32),
            jax.ShapeDtypeStruct((KH, S, HD), jnp.float32),
            jax.ShapeDtypeStruct((KH, S, HD), jnp.float32),
            jax.ShapeDtypeStruct((S, D), jnp.float32),
        ],
    )(cos_t, sin_t, x, r, ln1, qkv_w, qkv_b2)

    nq = S // BQ
    attn = pl.pallas_call(
        _attn_body,
        grid=(H, nq),
        in_specs=[
            pl.BlockSpec((1, H), lambda h, qi: (0, 0)),
            pl.BlockSpec((1, BQ, HD), lambda h, qi: (h, qi, 0)),
            pl.BlockSpec((1, S, HD), lambda h, qi: (h // 4, 0, 0)),
            pl.BlockSpec((1, S, HD), lambda h, qi: (h // 4, 0, 0)),
        ],
        out_specs=pl.BlockSpec((1, BQ, HD), lambda h, qi: (h, qi, 0)),
        out_shape=jax.ShapeDtypeStruct((H, S, HD), jnp.float32),
        scratch_shapes=[pltpu.VMEM((BQ, S), jnp.float32)],
    )(sinks2, q_rot, k_rot, v_all)

    n3 = S // BT3
    xflat, resid2, scores = pl.pallas_call(
        _post_body,
        grid=(n3,),
        in_specs=[
            pl.BlockSpec((H, BT3, HD), lambda i: (0, i, 0)),
            pl.BlockSpec((BT3, D), lambda i: (i, 0)),
            pl.BlockSpec((1, D), lambda i: (0, 0)),
            pl.BlockSpec((QW, D), lambda i: (0, 0)),
            pl.BlockSpec((1, D), lambda i: (0, 0)),
            pl.BlockSpec((D, 128), lambda i: (0, 0)),
            pl.BlockSpec((1, 128), lambda i: (0, 0)),
        ],
        out_specs=[
            pl.BlockSpec((BT3, D), lambda i: (i, 0)),
            pl.BlockSpec((BT3, D), lambda i: (i, 0)),
            pl.BlockSpec((BT3, 128), lambda i: (i, 0)),
        ],
        out_shape=[
            jax.ShapeDtypeStruct((S, D), jnp.float32),
            jax.ShapeDtypeStruct((S, D), jnp.float32),
            jax.ShapeDtypeStruct((S, 128), jnp.float32),
        ],
    )(attn, resid1, ln2, o_w, o_b2, rw_p, rb_p)

    n4 = S // BT4
    out = pl.pallas_call(
        _moe_body,
        grid=(n4, E),
        in_specs=[
            pl.BlockSpec((BT4, D), lambda t, e: (t, 0)),
            pl.BlockSpec((BT4, 128), lambda t, e: (t, 0)),
            pl.BlockSpec((1, D, 2 * F), lambda t, e: (e, 0, 0)),
            pl.BlockSpec((1, 1, 2 * F), lambda t, e: (e, 0, 0)),
            pl.BlockSpec((1, F, D), lambda t, e: (e, 0, 0)),
            pl.BlockSpec((1, 1, D), lambda t, e: (e, 0, 0)),
        ],
        out_specs=pl.BlockSpec((BT4, D), lambda t, e: (t, 0)),
        out_shape=jax.ShapeDtypeStruct((S, D), jnp.float32),
    )(xflat, scores, guw, gub, down_w, db3)

    return dict(out=out, resid2=resid2, xflat=xflat, scores=scores,
                attn=attn, resid1=resid1, q_rot=q_rot, k_rot=k_rot)


def kernel(hidden_states, residual, positions, ln1_w, ln2_w, qkv_w, qkv_b,
           o_w, o_b, sinks, router_w, router_b, gate_up_w, gate_up_b,
           down_w, down_b):
    p = _forward(hidden_states, residual, positions, ln1_w, ln2_w, qkv_w,
                 qkv_b, o_w, o_b, sinks, router_w, router_b, gate_up_w,
                 gate_up_b, down_w, down_b)
    return p["out"].reshape(B, S, D), p["resid2"].reshape(B, S, D)
